# baseline (speedup 1.0000x reference)
import functools

import jax
import jax.numpy as jnp
from jax import lax
from jax.experimental import pallas as pl
from jax.experimental.pallas import tpu as pltpu
from jax.experimental.pallas import tpu_sc as plsc

NC = 2
NS = 16
NW = NC * NS
CHUNK = 256


def _mesh():
    return plsc.VectorSubcoreMesh(
        core_axis_name="c", subcore_axis_name="s", num_cores=NC, num_subcores=NS
    )


def _stage_indices(edge_idx_hbm, row, flat_v, idx2_v, n_dummy, ew, ew_pad, wid,
                   k_chunks):
    fill0 = (ew // 16) * 16
    for i in range((ew_pad - fill0) // 16):
        flat_v[pl.ds(fill0 + i * 16, 16)] = jnp.full((16,), n_dummy, jnp.int32)
    pltpu.sync_copy(edge_idx_hbm.at[row, pl.ds(wid * ew, ew)],
                    flat_v.at[pl.ds(0, ew)])

    def rowcp(j, carry):
        for cidx in range(CHUNK // 16):
            idx2_v[j, pl.ds(cidx * 16, 16)] = flat_v[pl.ds(j * CHUNK + cidx * 16, 16)]
        return carry

    lax.fori_loop(0, k_chunks, rowcp, 0)


def _make_deg_kernel(n, n_pad, ew, k_chunks):
    rpt = n_pad // NS
    ew_pad = k_chunks * CHUNK
    WINDOW = 16

    def body(edge_idx, zeros1, out, acc, flat_v, dst_v, ones_v, zbuf, sem):
        c = lax.axis_index("c")
        s = lax.axis_index("s")
        wid = c * NS + s
        for i in range(CHUNK // 16):
            ones_v[pl.ds(i * 16, 16)] = jnp.ones((16,), jnp.float32)
        _stage_indices(edge_idx, 1, flat_v, dst_v, n, ew, ew_pad, wid, k_chunks)
        pltpu.sync_copy(zeros1.at[pl.ds(s * rpt, rpt)], zbuf)
        pltpu.sync_copy(zbuf, acc.at[pl.ds(s * rpt, rpt)])
        plsc.subcore_barrier()

        for j in range(k_chunks):
            pltpu.async_copy(ones_v, acc.at[dst_v.at[j]], sem, add=True)
            if j >= WINDOW:
                pltpu.make_async_copy(ones_v, acc.at[dst_v.at[0]], sem).wait()
        for _ in range(min(WINDOW, k_chunks)):
            pltpu.make_async_copy(ones_v, acc.at[dst_v.at[0]], sem).wait()
        plsc.subcore_barrier()
        pltpu.sync_copy(acc.at[pl.ds(s * rpt, rpt)], zbuf)
        pltpu.sync_copy(zbuf, out.at[pl.ds(c * n_pad + s * rpt, rpt)])

    return pl.kernel(
        body,
        out_type=jax.ShapeDtypeStruct((NC * n_pad,), jnp.float32),
        mesh=_mesh(),
        compiler_params=pltpu.CompilerParams(use_tc_tiling_on_sc=False),
        scratch_types=[
            pltpu.VMEM_SHARED((n_pad,), jnp.float32),
            pltpu.VMEM((ew_pad,), jnp.int32),
            pltpu.VMEM((k_chunks, CHUNK), jnp.int32),
            pltpu.VMEM((CHUNK,), jnp.float32),
            pltpu.VMEM((rpt,), jnp.float32),
            pltpu.SemaphoreType.DMA,
        ],
    )


NSLOT = 4


def _make_seg_sum_kernel(f, n, n_pad, ew, k_chunks):
    rpt = n_pad // NS
    ew_pad = k_chunks * CHUNK
    assert k_chunks % (2 * NSLOT) == 0
    n_rounds = k_chunks // NSLOT

    def body(hn, src2, dst2, zeros, out, acc, src_v, dst_v, *bufs_and_sems):
        rows = [[bufs_and_sems[2 * b + p] for p in range(2)] for b in range(NSLOT)]
        zbuf = bufs_and_sems[2 * NSLOT]
        gsem = bufs_and_sems[2 * NSLOT + 1: 2 * NSLOT + 1 + NSLOT]
        ssem = bufs_and_sems[2 * NSLOT + 1 + NSLOT:]
        c = lax.axis_index("c")
        s = lax.axis_index("s")
        wid = c * NS + s
        pltpu.sync_copy(src2.at[wid], src_v)
        pltpu.sync_copy(dst2.at[wid], dst_v)
        pltpu.sync_copy(zeros.at[pl.ds(s * rpt, rpt)], zbuf)
        pltpu.sync_copy(zbuf, acc.at[pl.ds(s * rpt, rpt)])
        plsc.subcore_barrier()

        def round_body(t, p, first, last):
            for b in range(NSLOT):
                j = t * NSLOT + b
                pltpu.make_async_copy(hn.at[src_v.at[j]], rows[b][p],
                                      gsem[b]).wait()
                pltpu.async_copy(rows[b][p], acc.at[dst_v.at[j]], ssem[b],
                                 add=True)
                if not first:
                    pltpu.make_async_copy(rows[b][1 - p], acc.at[dst_v.at[j]],
                                          ssem[b]).wait()
                if not last:
                    pltpu.async_copy(hn.at[src_v.at[j + NSLOT]], rows[b][1 - p],
                                     gsem[b])

        for b in range(NSLOT):
            pltpu.async_copy(hn.at[src_v.at[b]], rows[b][0], gsem[b])
        round_body(0, 0, True, False)

        def super_round(u, carry):
            round_body(2 * u + 1, 1, False, False)
            round_body(2 * u + 2, 0, False, False)
            return carry

        lax.fori_loop(0, (n_rounds - 2) // 2, super_round, 0)
        round_body(n_rounds - 1, 1, False, True)
        for b in range(NSLOT):
            pltpu.make_async_copy(rows[b][1], acc.at[dst_v.at[0]], ssem[b]).wait()

        plsc.subcore_barrier()
        pltpu.sync_copy(acc.at[pl.ds(s * rpt, rpt)], zbuf)
        pltpu.sync_copy(zbuf, out.at[c, pl.ds(s * rpt, rpt)])

    return pl.kernel(
        body,
        out_type=jax.ShapeDtypeStruct((NC, n_pad, f), jnp.float32),
        mesh=_mesh(),
        compiler_params=pltpu.CompilerParams(use_tc_tiling_on_sc=False),
        scratch_types=[
            pltpu.VMEM_SHARED((n_pad, f), jnp.float32),
            pltpu.VMEM((k_chunks, CHUNK), jnp.int32),
            pltpu.VMEM((k_chunks, CHUNK), jnp.int32),
        ]
        + [pltpu.VMEM((CHUNK, f), jnp.float32) for _ in range(2 * NSLOT)]
        + [pltpu.VMEM((rpt, f), jnp.float32)]
        + [pltpu.SemaphoreType.DMA for _ in range(2 * NSLOT)],
    )


def _mm_body(x_ref, w_ref, o_ref):
    o_ref[...] = jnp.dot(x_ref[...], w_ref[...], preferred_element_type=jnp.float32)


def _dinv_col(degt_ref):
    deg = degt_ref[:, 0:1] + degt_ref[:, 1:2] + 1.0
    return lax.rsqrt(deg)


def _scale_body(deg_ref, h_ref, o_ref):
    o_ref[...] = h_ref[...] * _dinv_col(deg_ref)


def _mid_body(deg_ref, s_ref, hn_ref, b_ref, w_ref, o_ref):
    dinv = _dinv_col(deg_ref)
    comb = (s_ref[0] + s_ref[1] + hn_ref[...]) * dinv + b_ref[...]
    a = jnp.maximum(comb, 0.0)
    h = jnp.dot(a, w_ref[...], preferred_element_type=jnp.float32)
    o_ref[...] = h * dinv


def _final_body(n, deg_ref, s_ref, hn_ref, b3_ref, dw1_ref, db1_ref,
                dw2_ref, db2_ref, dw3_ref, db3_ref, z_ref, xr_ref):
    dinv = _dinv_col(deg_ref)
    out3 = (s_ref[0] + s_ref[1] + hn_ref[...]) * dinv + b3_ref[...]
    z = jnp.mean(out3, axis=0, keepdims=True)
    d = jnp.maximum(jnp.dot(z, dw1_ref[...], preferred_element_type=jnp.float32)
                    + db1_ref[...], 0.0)
    d = jnp.maximum(jnp.dot(d, dw2_ref[...], preferred_element_type=jnp.float32)
                    + db2_ref[...], 0.0)
    row = jnp.dot(d, dw3_ref[...], preferred_element_type=jnp.float32) + db3_ref[...]
    z_ref[...] = z
    xr_ref[...] = jnp.broadcast_to(row, (n, row.shape[1]))


def kernel(x, edge_index, W1, b1, W2, b2, W3, b3, DW1, Db1, DW2, Db2, DW3, Db3):
    n, d_in = x.shape
    e = edge_index.shape[1]
    f1, f2, f3 = W1.shape[1], W2.shape[1], W3.shape[1]

    ew = e // NW
    k_chunks = -(-(-(-ew // CHUNK)) // (2 * NSLOT)) * (2 * NSLOT)
    n_pad = -(-(n + NS) // (NS * 8)) * (NS * 8)

    e_pad = NW * CHUNK * k_chunks
    pad = e_pad - e
    pad_ids = jnp.arange(pad, dtype=edge_index.dtype)
    src_p = jnp.concatenate([edge_index[0], pad_ids % 256])
    dst_p = jnp.concatenate([edge_index[1], n + (pad_ids % NS)])
    src2 = src_p.reshape(NW, k_chunks, CHUNK)
    dst2 = dst_p.reshape(NW, k_chunks, CHUNK)

    zeros1 = jnp.zeros((n_pad,), jnp.float32)
    zeros_f = {f: jnp.zeros((n_pad, f), jnp.float32) for f in {f1, f2, f3}}

    degp = _make_deg_kernel(n, n_pad, ew, k_chunks)(edge_index, zeros1)
    degt = degp.reshape(NC, n_pad).T

    blk = 2000 if n % 2000 == 0 else n
    grid = n // blk

    h1 = pl.pallas_call(
        _mm_body,
        grid=(grid,),
        in_specs=[
            pl.BlockSpec((blk, d_in), lambda i: (i, 0)),
            pl.BlockSpec((d_in, f1), lambda i: (0, 0)),
        ],
        out_specs=pl.BlockSpec((blk, f1), lambda i: (i, 0)),
        out_shape=jax.ShapeDtypeStruct((n, f1), jnp.float32),
    )(x, W1)

    hn1 = pl.pallas_call(
        _scale_body,
        grid=(grid,),
        in_specs=[
            pl.BlockSpec((blk, NC), lambda i: (i, 0)),
            pl.BlockSpec((blk, f1), lambda i: (i, 0)),
        ],
        out_specs=pl.BlockSpec((blk, f1), lambda i: (i, 0)),
        out_shape=jax.ShapeDtypeStruct((n, f1), jnp.float32),
    )(degt, h1)

    def mid(s_partial, hn, b_prev, w_next, f_in, f_out):
        return pl.pallas_call(
            _mid_body,
            grid=(grid,),
            in_specs=[
                pl.BlockSpec((blk, NC), lambda i: (i, 0)),
                pl.BlockSpec((NC, blk, f_in), lambda i: (0, i, 0)),
                pl.BlockSpec((blk, f_in), lambda i: (i, 0)),
                pl.BlockSpec((1, f_in), lambda i: (0, 0)),
                pl.BlockSpec((f_in, f_out), lambda i: (0, 0)),
            ],
            out_specs=pl.BlockSpec((blk, f_out), lambda i: (i, 0)),
            out_shape=jax.ShapeDtypeStruct((n, f_out), jnp.float32),
        )(degt, s_partial, hn, b_prev.reshape(1, f_in), w_next)

    s1 = _make_seg_sum_kernel(f1, n, n_pad, ew, k_chunks)(hn1, src2, dst2, zeros_f[f1])
    hn2 = mid(s1, hn1, b1, W2, f1, f2)
    s2 = _make_seg_sum_kernel(f2, n, n_pad, ew, k_chunks)(hn2, src2, dst2, zeros_f[f2])
    hn3 = mid(s2, hn2, b2, W3, f2, f3)
    s3 = _make_seg_sum_kernel(f3, n, n_pad, ew, k_chunks)(hn3, src2, dst2, zeros_f[f3])

    d_out = DW3.shape[1]
    z, x_recon = pl.pallas_call(
        functools.partial(_final_body, n),
        grid=(1,),
        in_specs=[
            pl.BlockSpec((n, NC), lambda i: (0, 0)),
            pl.BlockSpec((NC, n, f3), lambda i: (0, 0, 0)),
            pl.BlockSpec((n, f3), lambda i: (0, 0)),
            pl.BlockSpec((1, f3), lambda i: (0, 0)),
            pl.BlockSpec(DW1.shape, lambda i: (0, 0)),
            pl.BlockSpec((1, DW1.shape[1]), lambda i: (0, 0)),
            pl.BlockSpec(DW2.shape, lambda i: (0, 0)),
            pl.BlockSpec((1, DW2.shape[1]), lambda i: (0, 0)),
            pl.BlockSpec(DW3.shape, lambda i: (0, 0)),
            pl.BlockSpec((1, d_out), lambda i: (0, 0)),
        ],
        out_specs=[
            pl.BlockSpec((1, f3), lambda i: (0, 0)),
            pl.BlockSpec((n, d_out), lambda i: (0, 0)),
        ],
        out_shape=[
            jax.ShapeDtypeStruct((1, f3), jnp.float32),
            jax.ShapeDtypeStruct((n, d_out), jnp.float32),
        ],
    )(degt, s3, hn3, b3.reshape(1, f3),
      DW1, Db1.reshape(1, -1),
      DW2, Db2.reshape(1, -1), DW3, Db3.reshape(1, -1))
    return (z, x_recon)


# speedup vs baseline: 1.0078x; 1.0078x over previous
import functools

import jax
import jax.numpy as jnp
from jax import lax
from jax.experimental import pallas as pl
from jax.experimental.pallas import tpu as pltpu
from jax.experimental.pallas import tpu_sc as plsc

NC = 2
NS = 16
NW = NC * NS
CHUNK = 128


def _mesh():
    return plsc.VectorSubcoreMesh(
        core_axis_name="c", subcore_axis_name="s", num_cores=NC, num_subcores=NS
    )


def _stage_indices(edge_idx_hbm, row, flat_v, idx2_v, n_dummy, ew, ew_pad, wid,
                   k_chunks):
    fill0 = (ew // 16) * 16
    for i in range((ew_pad - fill0) // 16):
        flat_v[pl.ds(fill0 + i * 16, 16)] = jnp.full((16,), n_dummy, jnp.int32)
    pltpu.sync_copy(edge_idx_hbm.at[row, pl.ds(wid * ew, ew)],
                    flat_v.at[pl.ds(0, ew)])

    def rowcp(j, carry):
        for cidx in range(CHUNK // 16):
            idx2_v[j, pl.ds(cidx * 16, 16)] = flat_v[pl.ds(j * CHUNK + cidx * 16, 16)]
        return carry

    lax.fori_loop(0, k_chunks, rowcp, 0)


def _make_deg_kernel(n, n_pad, ew, k_chunks):
    rpt = n_pad // NS
    ew_pad = k_chunks * CHUNK
    WINDOW = 16

    def body(edge_idx, zeros1, out, acc, flat_v, dst_v, ones_v, zbuf, sem):
        c = lax.axis_index("c")
        s = lax.axis_index("s")
        wid = c * NS + s
        for i in range(CHUNK // 16):
            ones_v[pl.ds(i * 16, 16)] = jnp.ones((16,), jnp.float32)
        _stage_indices(edge_idx, 1, flat_v, dst_v, n, ew, ew_pad, wid, k_chunks)
        pltpu.sync_copy(zeros1.at[pl.ds(s * rpt, rpt)], zbuf)
        pltpu.sync_copy(zbuf, acc.at[pl.ds(s * rpt, rpt)])
        plsc.subcore_barrier()

        for j in range(k_chunks):
            pltpu.async_copy(ones_v, acc.at[dst_v.at[j]], sem, add=True)
            if j >= WINDOW:
                pltpu.make_async_copy(ones_v, acc.at[dst_v.at[0]], sem).wait()
        for _ in range(min(WINDOW, k_chunks)):
            pltpu.make_async_copy(ones_v, acc.at[dst_v.at[0]], sem).wait()
        plsc.subcore_barrier()
        pltpu.sync_copy(acc.at[pl.ds(s * rpt, rpt)], zbuf)
        pltpu.sync_copy(zbuf, out.at[pl.ds(c * n_pad + s * rpt, rpt)])

    return pl.kernel(
        body,
        out_type=jax.ShapeDtypeStruct((NC * n_pad,), jnp.float32),
        mesh=_mesh(),
        compiler_params=pltpu.CompilerParams(use_tc_tiling_on_sc=False),
        scratch_types=[
            pltpu.VMEM_SHARED((n_pad,), jnp.float32),
            pltpu.VMEM((ew_pad,), jnp.int32),
            pltpu.VMEM((k_chunks, CHUNK), jnp.int32),
            pltpu.VMEM((CHUNK,), jnp.float32),
            pltpu.VMEM((rpt,), jnp.float32),
            pltpu.SemaphoreType.DMA,
        ],
    )


NSLOT = 8


def _make_seg_sum_kernel(f, n, n_pad, ew, k_chunks):
    rpt = n_pad // NS
    ew_pad = k_chunks * CHUNK
    assert k_chunks % (2 * NSLOT) == 0
    n_rounds = k_chunks // NSLOT

    def body(hn, src2, dst2, zeros, out, acc, src_v, dst_v, *bufs_and_sems):
        rows = [[bufs_and_sems[2 * b + p] for p in range(2)] for b in range(NSLOT)]
        zbuf = bufs_and_sems[2 * NSLOT]
        gsem = bufs_and_sems[2 * NSLOT + 1: 2 * NSLOT + 1 + NSLOT]
        ssem = bufs_and_sems[2 * NSLOT + 1 + NSLOT:]
        c = lax.axis_index("c")
        s = lax.axis_index("s")
        wid = c * NS + s
        pltpu.sync_copy(src2.at[wid], src_v)
        pltpu.sync_copy(dst2.at[wid], dst_v)
        pltpu.sync_copy(zeros.at[pl.ds(s * rpt, rpt)], zbuf)
        pltpu.sync_copy(zbuf, acc.at[pl.ds(s * rpt, rpt)])
        plsc.subcore_barrier()

        def round_body(t, p, first, last):
            for b in range(NSLOT):
                j = t * NSLOT + b
                pltpu.make_async_copy(hn.at[src_v.at[j]], rows[b][p],
                                      gsem[b]).wait()
                pltpu.async_copy(rows[b][p], acc.at[dst_v.at[j]], ssem[b],
                                 add=True)
                if not first:
                    pltpu.make_async_copy(rows[b][1 - p], acc.at[dst_v.at[j]],
                                          ssem[b]).wait()
                if not last:
                    pltpu.async_copy(hn.at[src_v.at[j + NSLOT]], rows[b][1 - p],
                                     gsem[b])

        for b in range(NSLOT):
            pltpu.async_copy(hn.at[src_v.at[b]], rows[b][0], gsem[b])
        round_body(0, 0, True, False)

        def super_round(u, carry):
            round_body(2 * u + 1, 1, False, False)
            round_body(2 * u + 2, 0, False, False)
            return carry

        lax.fori_loop(0, (n_rounds - 2) // 2, super_round, 0)
        round_body(n_rounds - 1, 1, False, True)
        for b in range(NSLOT):
            pltpu.make_async_copy(rows[b][1], acc.at[dst_v.at[0]], ssem[b]).wait()

        plsc.subcore_barrier()
        pltpu.sync_copy(acc.at[pl.ds(s * rpt, rpt)], zbuf)
        pltpu.sync_copy(zbuf, out.at[c, pl.ds(s * rpt, rpt)])

    return pl.kernel(
        body,
        out_type=jax.ShapeDtypeStruct((NC, n_pad, f), jnp.float32),
        mesh=_mesh(),
        compiler_params=pltpu.CompilerParams(use_tc_tiling_on_sc=False),
        scratch_types=[
            pltpu.VMEM_SHARED((n_pad, f), jnp.float32),
            pltpu.VMEM((k_chunks, CHUNK), jnp.int32),
            pltpu.VMEM((k_chunks, CHUNK), jnp.int32),
        ]
        + [pltpu.VMEM((CHUNK, f), jnp.float32) for _ in range(2 * NSLOT)]
        + [pltpu.VMEM((rpt, f), jnp.float32)]
        + [pltpu.SemaphoreType.DMA for _ in range(2 * NSLOT)],
    )


def _mm_body(x_ref, w_ref, o_ref):
    o_ref[...] = jnp.dot(x_ref[...], w_ref[...], preferred_element_type=jnp.float32)


def _dinv_col(degt_ref):
    deg = degt_ref[:, 0:1] + degt_ref[:, 1:2] + 1.0
    return lax.rsqrt(deg)


def _scale_body(deg_ref, h_ref, o_ref):
    o_ref[...] = h_ref[...] * _dinv_col(deg_ref)


def _mid_body(deg_ref, s_ref, hn_ref, b_ref, w_ref, o_ref):
    dinv = _dinv_col(deg_ref)
    comb = (s_ref[0] + s_ref[1] + hn_ref[...]) * dinv + b_ref[...]
    a = jnp.maximum(comb, 0.0)
    h = jnp.dot(a, w_ref[...], preferred_element_type=jnp.float32)
    o_ref[...] = h * dinv


def _final_body(n, deg_ref, s_ref, hn_ref, b3_ref, dw1_ref, db1_ref,
                dw2_ref, db2_ref, dw3_ref, db3_ref, z_ref, xr_ref):
    dinv = _dinv_col(deg_ref)
    out3 = (s_ref[0] + s_ref[1] + hn_ref[...]) * dinv + b3_ref[...]
    z = jnp.mean(out3, axis=0, keepdims=True)
    d = jnp.maximum(jnp.dot(z, dw1_ref[...], preferred_element_type=jnp.float32)
                    + db1_ref[...], 0.0)
    d = jnp.maximum(jnp.dot(d, dw2_ref[...], preferred_element_type=jnp.float32)
                    + db2_ref[...], 0.0)
    row = jnp.dot(d, dw3_ref[...], preferred_element_type=jnp.float32) + db3_ref[...]
    z_ref[...] = z
    xr_ref[...] = jnp.broadcast_to(row, (n, row.shape[1]))


def kernel(x, edge_index, W1, b1, W2, b2, W3, b3, DW1, Db1, DW2, Db2, DW3, Db3):
    n, d_in = x.shape
    e = edge_index.shape[1]
    f1, f2, f3 = W1.shape[1], W2.shape[1], W3.shape[1]

    ew = e // NW
    k_chunks = -(-(-(-ew // CHUNK)) // (2 * NSLOT)) * (2 * NSLOT)
    n_pad = -(-(n + NS) // (NS * 8)) * (NS * 8)

    e_pad = NW * CHUNK * k_chunks
    pad = e_pad - e
    pad_ids = jnp.arange(pad, dtype=edge_index.dtype)
    src_p = jnp.concatenate([edge_index[0], pad_ids % 256])
    dst_p = jnp.concatenate([edge_index[1], n + (pad_ids % NS)])
    src2 = src_p.reshape(NW, k_chunks, CHUNK)
    dst2 = dst_p.reshape(NW, k_chunks, CHUNK)

    zeros1 = jnp.zeros((n_pad,), jnp.float32)
    zeros_f = {f: jnp.zeros((n_pad, f), jnp.float32) for f in {f1, f2, f3}}

    degp = _make_deg_kernel(n, n_pad, ew, k_chunks)(edge_index, zeros1)
    degt = degp.reshape(NC, n_pad).T

    blk = 2000 if n % 2000 == 0 else n
    grid = n // blk

    h1 = pl.pallas_call(
        _mm_body,
        grid=(grid,),
        in_specs=[
            pl.BlockSpec((blk, d_in), lambda i: (i, 0)),
            pl.BlockSpec((d_in, f1), lambda i: (0, 0)),
        ],
        out_specs=pl.BlockSpec((blk, f1), lambda i: (i, 0)),
        out_shape=jax.ShapeDtypeStruct((n, f1), jnp.float32),
    )(x, W1)

    hn1 = pl.pallas_call(
        _scale_body,
        grid=(grid,),
        in_specs=[
            pl.BlockSpec((blk, NC), lambda i: (i, 0)),
            pl.BlockSpec((blk, f1), lambda i: (i, 0)),
        ],
        out_specs=pl.BlockSpec((blk, f1), lambda i: (i, 0)),
        out_shape=jax.ShapeDtypeStruct((n, f1), jnp.float32),
    )(degt, h1)

    def mid(s_partial, hn, b_prev, w_next, f_in, f_out):
        return pl.pallas_call(
            _mid_body,
            grid=(grid,),
            in_specs=[
                pl.BlockSpec((blk, NC), lambda i: (i, 0)),
                pl.BlockSpec((NC, blk, f_in), lambda i: (0, i, 0)),
                pl.BlockSpec((blk, f_in), lambda i: (i, 0)),
                pl.BlockSpec((1, f_in), lambda i: (0, 0)),
                pl.BlockSpec((f_in, f_out), lambda i: (0, 0)),
            ],
            out_specs=pl.BlockSpec((blk, f_out), lambda i: (i, 0)),
            out_shape=jax.ShapeDtypeStruct((n, f_out), jnp.float32),
        )(degt, s_partial, hn, b_prev.reshape(1, f_in), w_next)

    s1 = _make_seg_sum_kernel(f1, n, n_pad, ew, k_chunks)(hn1, src2, dst2, zeros_f[f1])
    hn2 = mid(s1, hn1, b1, W2, f1, f2)
    s2 = _make_seg_sum_kernel(f2, n, n_pad, ew, k_chunks)(hn2, src2, dst2, zeros_f[f2])
    hn3 = mid(s2, hn2, b2, W3, f2, f3)
    s3 = _make_seg_sum_kernel(f3, n, n_pad, ew, k_chunks)(hn3, src2, dst2, zeros_f[f3])

    d_out = DW3.shape[1]
    z, x_recon = pl.pallas_call(
        functools.partial(_final_body, n),
        grid=(1,),
        in_specs=[
            pl.BlockSpec((n, NC), lambda i: (0, 0)),
            pl.BlockSpec((NC, n, f3), lambda i: (0, 0, 0)),
            pl.BlockSpec((n, f3), lambda i: (0, 0)),
            pl.BlockSpec((1, f3), lambda i: (0, 0)),
            pl.BlockSpec(DW1.shape, lambda i: (0, 0)),
            pl.BlockSpec((1, DW1.shape[1]), lambda i: (0, 0)),
            pl.BlockSpec(DW2.shape, lambda i: (0, 0)),
            pl.BlockSpec((1, DW2.shape[1]), lambda i: (0, 0)),
            pl.BlockSpec(DW3.shape, lambda i: (0, 0)),
            pl.BlockSpec((1, d_out), lambda i: (0, 0)),
        ],
        out_specs=[
            pl.BlockSpec((1, f3), lambda i: (0, 0)),
            pl.BlockSpec((n, d_out), lambda i: (0, 0)),
        ],
        out_shape=[
            jax.ShapeDtypeStruct((1, f3), jnp.float32),
            jax.ShapeDtypeStruct((n, d_out), jnp.float32),
        ],
    )(degt, s3, hn3, b3.reshape(1, f3),
      DW1, Db1.reshape(1, -1),
      DW2, Db2.reshape(1, -1), DW3, Db3.reshape(1, -1))
    return (z, x_recon)
